# baseline (speedup 1.0000x reference)
import jax
import jax.numpy as jnp
from jax import lax
from jax.experimental import pallas as pl
from jax.experimental.pallas import tpu as pltpu
from jax.experimental.pallas import tpu_sc as plsc

_EMBED = 64
_NC = 2
_NS = 16
_NW = _NC * _NS
_NBUF = 4


def _sc_gather(idx_hbm, table_hbm, out_hbm, idx_v, rows, gsem, wsem):
    cpw = idx_hbm.shape[1]
    wid = lax.axis_index("s") * _NC + lax.axis_index("c")
    rbase = wid * cpw
    pltpu.sync_copy(idx_hbm.at[wid], idx_v)

    def gstart(j, b):
        pltpu.async_copy(table_hbm.at[idx_v.at[j]], rows.at[b], gsem.at[b])

    def gwait(j, b):
        pltpu.make_async_copy(
            table_hbm.at[idx_v.at[j]], rows.at[b], gsem.at[b]).wait()

    def wstart(j, b):
        pltpu.async_copy(rows.at[b], out_hbm.at[rbase + j], wsem.at[b])

    def wwait(j, b):
        pltpu.make_async_copy(
            rows.at[b], out_hbm.at[rbase + j], wsem.at[b]).wait()

    for b in range(_NBUF):
        gstart(b, b)

    nsteps = cpw // _NBUF

    def body(step, carry):
        base = step * _NBUF
        for b in range(_NBUF):
            gwait(base + b, b)
            wstart(base + b, b)
        for b in range(_NBUF):
            wwait(base + b, b)
            gstart(base + _NBUF + b, b)
        return carry

    lax.fori_loop(0, nsteps - 1, body, 0)
    tail = (nsteps - 1) * _NBUF
    for b in range(_NBUF):
        gwait(tail + b, b)
        wstart(tail + b, b)
    for b in range(_NBUF):
        wwait(tail + b, b)


def kernel(x, encoding):
    n, s = x.shape
    cpw = n // _NW
    idx = x.reshape(_NW, cpw, s).astype(jnp.int32)
    out = pl.kernel(
        _sc_gather,
        out_type=jax.ShapeDtypeStruct((n, s, _EMBED), jnp.float32),
        mesh=plsc.VectorSubcoreMesh(core_axis_name="c", subcore_axis_name="s"),
        compiler_params=pltpu.CompilerParams(use_tc_tiling_on_sc=False),
        scratch_types=[
            pltpu.VMEM((cpw, s), jnp.int32),
            pltpu.VMEM((_NBUF, s, _EMBED), jnp.float32),
            pltpu.SemaphoreType.DMA((_NBUF,)),
            pltpu.SemaphoreType.DMA((_NBUF,)),
        ],
    )(idx, encoding)
    return out
